# baseline (device time: 20605 ns/iter reference)
import os

import jax
import jax.numpy as jnp
from jax import lax
from jax.experimental import pallas as pl
from jax.experimental.pallas import tpu as pltpu

N_DEV = 4
BLOCK_M = int(os.environ.get("KBLOCK", "512"))

_MODE = os.environ.get("KMODE", "full")


def kernel(x, dy, gamma):
    del gamma
    m, d = x.shape
    n_blocks = m // BLOCK_M

    def body(x_ref, dy_ref, out_ref, acc_ref, comm_ref, send_sems, recv_sems):
        i = pl.program_id(0)

        @pl.when(i == 0)
        def _():
            acc_ref[...] = jnp.zeros_like(acc_ref)

        @pl.when((i == 0) & (_MODE == "full"))
        def _():
            my_pos = lax.axis_index("i")
            barrier_sem = pltpu.get_barrier_semaphore()
            for k in range(1, N_DEV):
                pl.semaphore_signal(
                    barrier_sem, inc=1,
                    device_id=((my_pos + k) % N_DEV,),
                    device_id_type=pl.DeviceIdType.MESH,
                )
            pl.semaphore_wait(barrier_sem, N_DEV - 1)

        xb = x_ref[...]
        dyb = dy_ref[...]
        if _MODE == "dma_only":
            acc_ref[...] += jnp.concatenate(
                [jnp.sum(xb, axis=0, keepdims=True),
                 jnp.sum(dyb, axis=0, keepdims=True)], axis=0)
        else:
            mu = jnp.mean(xb, axis=1, keepdims=True)
            xc = xb - mu
            var = jnp.mean(xc * xc, axis=1, keepdims=True)
            rstd = lax.rsqrt(var + 1e-5)
            xhat = xc * rstd
            dgamma = jnp.sum(dyb * xhat, axis=0, keepdims=True)
            dbeta = jnp.sum(dyb, axis=0, keepdims=True)
            acc_ref[...] += jnp.concatenate([dgamma, dbeta], axis=0)

        @pl.when((i == n_blocks - 1) & (_MODE == "full"))
        def _():
            my_pos = lax.axis_index("i")

            comm_ref[my_pos] = acc_ref[...]
            sends = []
            for k in (2, 1, 3):
                peer = (my_pos + k) % N_DEV
                send = pltpu.make_async_remote_copy(
                    src_ref=comm_ref.at[my_pos],
                    dst_ref=comm_ref.at[my_pos],
                    send_sem=send_sems.at[k - 1],
                    recv_sem=recv_sems.at[my_pos],
                    device_id=(peer,),
                    device_id_type=pl.DeviceIdType.MESH,
                )
                send.start()
                sends.append(send)

            out_ref[...] = acc_ref[...]
            for k in (1, 3, 2):
                src = (my_pos + k) % N_DEV
                recv = pltpu.make_async_remote_copy(
                    src_ref=comm_ref.at[src],
                    dst_ref=comm_ref.at[src],
                    send_sem=send_sems.at[k - 1],
                    recv_sem=recv_sems.at[src],
                    device_id=(my_pos,),
                    device_id_type=pl.DeviceIdType.MESH,
                )
                recv.wait_recv()
                out_ref[...] += comm_ref[src]
            for send in sends:
                send.wait_send()

    return pl.pallas_call(
        body,
        grid=(n_blocks,),
        in_specs=[
            pl.BlockSpec((BLOCK_M, d), lambda i: (i, 0)),
            pl.BlockSpec((BLOCK_M, d), lambda i: (i, 0)),
        ],
        out_specs=pl.BlockSpec((2, d), lambda i: (0, 0)),
        out_shape=jax.ShapeDtypeStruct((2, d), jnp.float32),
        scratch_shapes=[
            pltpu.VMEM((2, d), jnp.float32),
            pltpu.VMEM((N_DEV, 2, d), jnp.float32),
            pltpu.SemaphoreType.DMA((N_DEV - 1,)),
            pltpu.SemaphoreType.DMA((N_DEV,)),
        ],
        compiler_params=pltpu.CompilerParams(collective_id=0),
    )(x, dy)


# device time: 20584 ns/iter; 1.0010x vs baseline; 1.0010x over previous
import os

import jax
import jax.numpy as jnp
from jax import lax
from jax.experimental import pallas as pl
from jax.experimental.pallas import tpu as pltpu

N_DEV = 4
BLOCK_M = int(os.environ.get("KBLOCK", "512"))

_MODE = os.environ.get("KMODE", "full")


def kernel(x, dy, gamma):
    del gamma
    m, d = x.shape
    n_blocks = m // BLOCK_M

    def body(x_ref, dy_ref, out_ref, acc_ref, comm_ref, send_sems, recv_sems):
        i = pl.program_id(0)

        @pl.when(i == 0)
        def _():
            acc_ref[...] = jnp.zeros_like(acc_ref)

        @pl.when((i == 0) & (_MODE == "full"))
        def _():
            my_pos = lax.axis_index("i")
            barrier_sem = pltpu.get_barrier_semaphore()
            for k in range(1, N_DEV):
                pl.semaphore_signal(
                    barrier_sem, inc=1,
                    device_id=((my_pos + k) % N_DEV,),
                    device_id_type=pl.DeviceIdType.MESH,
                )
            pl.semaphore_wait(barrier_sem, N_DEV - 1)

        xb = x_ref[...]
        dyb = dy_ref[...]
        if _MODE == "dma_only":
            acc_ref[0:1, :] += jnp.sum(xb, axis=0, keepdims=True)
            acc_ref[1:2, :] += jnp.sum(dyb, axis=0, keepdims=True)
        else:
            mu = jnp.mean(xb, axis=1, keepdims=True)
            xc = xb - mu
            var = jnp.mean(xc * xc, axis=1, keepdims=True)
            rstd = lax.rsqrt(var + 1e-5)
            xhat = xc * rstd
            acc_ref[0:1, :] += jnp.sum(dyb * xhat, axis=0, keepdims=True)
            acc_ref[1:2, :] += jnp.sum(dyb, axis=0, keepdims=True)

        @pl.when((i == n_blocks - 1) & (_MODE == "full"))
        def _():
            my_pos = lax.axis_index("i")

            comm_ref[my_pos] = acc_ref[...]
            sends = []
            for k in (2, 1, 3):
                peer = (my_pos + k) % N_DEV
                send = pltpu.make_async_remote_copy(
                    src_ref=comm_ref.at[my_pos],
                    dst_ref=comm_ref.at[my_pos],
                    send_sem=send_sems.at[k - 1],
                    recv_sem=recv_sems.at[my_pos],
                    device_id=(peer,),
                    device_id_type=pl.DeviceIdType.MESH,
                )
                send.start()
                sends.append(send)

            out_ref[...] = acc_ref[...]
            for k in (1, 3, 2):
                src = (my_pos + k) % N_DEV
                recv = pltpu.make_async_remote_copy(
                    src_ref=comm_ref.at[src],
                    dst_ref=comm_ref.at[src],
                    send_sem=send_sems.at[k - 1],
                    recv_sem=recv_sems.at[src],
                    device_id=(my_pos,),
                    device_id_type=pl.DeviceIdType.MESH,
                )
                recv.wait_recv()
                out_ref[...] += comm_ref[src]
            for send in sends:
                send.wait_send()

    return pl.pallas_call(
        body,
        grid=(n_blocks,),
        in_specs=[
            pl.BlockSpec((BLOCK_M, d), lambda i: (i, 0)),
            pl.BlockSpec((BLOCK_M, d), lambda i: (i, 0)),
        ],
        out_specs=pl.BlockSpec((2, d), lambda i: (0, 0)),
        out_shape=jax.ShapeDtypeStruct((2, d), jnp.float32),
        scratch_shapes=[
            pltpu.VMEM((2, d), jnp.float32),
            pltpu.VMEM((N_DEV, 2, d), jnp.float32),
            pltpu.SemaphoreType.DMA((N_DEV - 1,)),
            pltpu.SemaphoreType.DMA((N_DEV,)),
        ],
        compiler_params=pltpu.CompilerParams(collective_id=0),
    )(x, dy)


# device time: 17409 ns/iter; 1.1836x vs baseline; 1.1824x over previous
import os

import jax
import jax.numpy as jnp
from jax import lax
from jax.experimental import pallas as pl
from jax.experimental.pallas import tpu as pltpu

N_DEV = 4
CHUNK_M = int(os.environ.get("KBLOCK", "256"))

_MODE = os.environ.get("KMODE", "full")


def kernel(x, dy, gamma):
    del gamma
    m, d = x.shape
    n_chunks = m // CHUNK_M

    def body(x_hbm, dy_hbm, out_ref, xv, dyv, comm_ref,
             in_sems, send_sems, recv_sems):
        my_pos = lax.axis_index("i")

        copies = []
        for c in range(n_chunks):
            rows = pl.ds(c * CHUNK_M, CHUNK_M)
            cx = pltpu.make_async_copy(
                x_hbm.at[rows, :], xv.at[rows, :], in_sems.at[0, c])
            cy = pltpu.make_async_copy(
                dy_hbm.at[rows, :], dyv.at[rows, :], in_sems.at[1, c])
            cx.start()
            cy.start()
            copies.append((cx, cy))

        if _MODE == "full":
            barrier_sem = pltpu.get_barrier_semaphore()
            for k in range(1, N_DEV):
                pl.semaphore_signal(
                    barrier_sem, inc=1,
                    device_id=((my_pos + k) % N_DEV,),
                    device_id_type=pl.DeviceIdType.MESH,
                )
            pl.semaphore_wait(barrier_sem, N_DEV - 1)

        dgamma = jnp.zeros((1, d), jnp.float32)
        dbeta = jnp.zeros((1, d), jnp.float32)
        for c, (cx, cy) in enumerate(copies):
            cx.wait()
            cy.wait()
            rows = pl.ds(c * CHUNK_M, CHUNK_M)
            xb = xv[rows, :]
            dyb = dyv[rows, :]
            if _MODE == "dma_only":
                dgamma = dgamma + jnp.sum(xb, axis=0, keepdims=True)
                dbeta = dbeta + jnp.sum(dyb, axis=0, keepdims=True)
            else:
                mu = jnp.mean(xb, axis=1, keepdims=True)
                xc = xb - mu
                var = jnp.mean(xc * xc, axis=1, keepdims=True)
                rstd = lax.rsqrt(var + 1e-5)
                xhat = xc * rstd
                dgamma = dgamma + jnp.sum(dyb * xhat, axis=0, keepdims=True)
                dbeta = dbeta + jnp.sum(dyb, axis=0, keepdims=True)

        out_ref[0:1, :] = dgamma
        out_ref[1:2, :] = dbeta

        if _MODE == "full":
            comm_ref[my_pos, 0:1, :] = dgamma
            comm_ref[my_pos, 1:2, :] = dbeta
            sends = []
            for k in (2, 1, 3):
                peer = (my_pos + k) % N_DEV
                send = pltpu.make_async_remote_copy(
                    src_ref=comm_ref.at[my_pos],
                    dst_ref=comm_ref.at[my_pos],
                    send_sem=send_sems.at[k - 1],
                    recv_sem=recv_sems.at[my_pos],
                    device_id=(peer,),
                    device_id_type=pl.DeviceIdType.MESH,
                )
                send.start()
                sends.append(send)

            for k in (1, 3, 2):
                src = (my_pos + k) % N_DEV
                recv = pltpu.make_async_remote_copy(
                    src_ref=comm_ref.at[src],
                    dst_ref=comm_ref.at[src],
                    send_sem=send_sems.at[k - 1],
                    recv_sem=recv_sems.at[src],
                    device_id=(my_pos,),
                    device_id_type=pl.DeviceIdType.MESH,
                )
                recv.wait_recv()
                out_ref[...] += comm_ref[src]
            for send in sends:
                send.wait_send()

    return pl.pallas_call(
        body,
        in_specs=[
            pl.BlockSpec(memory_space=pl.ANY),
            pl.BlockSpec(memory_space=pl.ANY),
        ],
        out_specs=pl.BlockSpec(memory_space=pltpu.VMEM),
        out_shape=jax.ShapeDtypeStruct((2, d), jnp.float32),
        scratch_shapes=[
            pltpu.VMEM((m, d), jnp.float32),
            pltpu.VMEM((m, d), jnp.float32),
            pltpu.VMEM((N_DEV, 2, d), jnp.float32),
            pltpu.SemaphoreType.DMA((2, m // CHUNK_M)),
            pltpu.SemaphoreType.DMA((N_DEV - 1,)),
            pltpu.SemaphoreType.DMA((N_DEV,)),
        ],
        compiler_params=pltpu.CompilerParams(
            collective_id=0,
            vmem_limit_bytes=48 * 1024 * 1024,
        ),
    )(x, dy)
